# baseline (speedup 1.0000x reference)
import functools

import jax
import jax.numpy as jnp
from jax import lax
from jax.experimental import pallas as pl
from jax.experimental.pallas import tpu as pltpu
from jax.experimental.pallas import tpu_sc as plsc

NUM_CORES = 2
NUM_SUBCORES = 16
NUM_WORKERS = NUM_CORES * NUM_SUBCORES
LANES = 16


def _embed_body(total_rows, dim, seq_len, chunk, seq_hbm, tok_hbm, pos_hbm,
                out_hbm, idx_v, rows_v, pos_v, sem):
  rows_per_w = total_rows // NUM_WORKERS
  nchunks = rows_per_w // chunk
  vregs_per_row = dim // LANES

  wid = lax.axis_index("s") * NUM_CORES + lax.axis_index("c")
  base = wid * rows_per_w

  def chunk_body(c, _):
    row0 = base + c * chunk
    pos0 = lax.rem(row0, seq_len)
    pltpu.sync_copy(seq_hbm.at[pl.ds(row0, chunk)], idx_v)
    gather = pltpu.async_copy(tok_hbm.at[idx_v], rows_v, sem)
    pltpu.sync_copy(pos_hbm.at[pl.ds(pos0, chunk)], pos_v)
    gather.wait()

    def row_body(r, _):
      def vec_body(v, _):
        sl = pl.ds(v * LANES, LANES)
        rows_v[r, sl] = rows_v[r, sl] + pos_v[r, sl]
        return 0

      lax.fori_loop(0, vregs_per_row, vec_body, 0)
      return 0

    lax.fori_loop(0, chunk, row_body, 0)
    pltpu.sync_copy(rows_v, out_hbm.at[pl.ds(row0, chunk)])
    return 0

  lax.fori_loop(0, nchunks, chunk_body, 0)


@functools.partial(jax.jit, static_argnames=("total_rows", "dim", "seq_len"))
def _embed(seq_flat, token_table, pos_table, *, total_rows, dim, seq_len):
  chunk = 32
  mesh = plsc.VectorSubcoreMesh(core_axis_name="c", subcore_axis_name="s")
  kfn = pl.kernel(
      functools.partial(_embed_body, total_rows, dim, seq_len, chunk),
      mesh=mesh,
      out_type=jax.ShapeDtypeStruct((total_rows, dim), jnp.float32),
      scratch_types=[
          pltpu.VMEM((chunk,), jnp.int32),
          pltpu.VMEM((chunk, dim), jnp.float32),
          pltpu.VMEM((chunk, dim), jnp.float32),
          pltpu.SemaphoreType.DMA,
      ],
  )
  return kfn(seq_flat, token_table, pos_table)


def kernel(sequence, token_table, pos_table):
  b, s = sequence.shape
  dim = token_table.shape[1]
  seq_flat = sequence.reshape(-1).astype(jnp.int32)
  out = _embed(seq_flat, token_table, pos_table,
               total_rows=b * s, dim=dim, seq_len=s)
  return out.reshape(b, s, dim)


# speedup vs baseline: 2.3866x; 2.3866x over previous
import functools

import jax
import jax.numpy as jnp
from jax import lax
from jax.experimental import pallas as pl
from jax.experimental.pallas import tpu as pltpu
from jax.experimental.pallas import tpu_sc as plsc

NUM_CORES = 2
NUM_SUBCORES = 16
NUM_WORKERS = NUM_CORES * NUM_SUBCORES
LANES = 16
CHUNK = 8
NBUF = 3


def _embed_body(nbatch, seqlen, dim, seq_hbm, tok_hbm, pos_hbm, out_hbm,
                idx_v, pos_v, tok_v, *sems):
  gs = sems[:NBUF]
  ws = sems[NBUF:]
  pw = seqlen // NUM_WORKERS
  nch = pw // CHUNK
  nvec = dim // LANES

  wid = lax.axis_index("s") * NUM_CORES + lax.axis_index("c")
  p0 = pl.multiple_of(wid * pw, pw)

  for b in range(nbatch):
    pltpu.sync_copy(seq_hbm.at[b, pl.ds(p0, pw)], idx_v.at[b])

  def issue_gathers(k, c):
    cps = [pltpu.async_copy(pos_hbm.at[pl.ds(p0 + c * CHUNK, CHUNK)],
                            pos_v.at[k], gs[k])]
    for b in range(nbatch):
      cps.append(pltpu.async_copy(
          tok_hbm.at[idx_v.at[b, pl.ds(c * CHUNK, CHUNK)]],
          tok_v.at[k, b], gs[k]))
    return cps

  def issue_writes(k, c):
    return [pltpu.async_copy(tok_v.at[k, b],
                             out_hbm.at[b, pl.ds(p0 + c * CHUNK, CHUNK)],
                             ws[k])
            for b in range(nbatch)]

  def add_pos(k):
    def row_body(r, carry):
      for v in range(nvec):
        sl = pl.ds(v * LANES, LANES)
        p = pos_v[k, r, sl]
        for b in range(nbatch):
          tok_v[k, b, r, sl] = tok_v[k, b, r, sl] + p
      return carry
    lax.fori_loop(0, CHUNK, row_body, 0)

  pend_g = {0: issue_gathers(0, 0)}
  if nch > 1:
    pend_g[1] = issue_gathers(1, 1)
  pend_w = {}
  for c in range(nch):
    k = c % NBUF
    for cp in pend_g.pop(k):
      cp.wait()
    add_pos(k)
    pend_w[k] = issue_writes(k, c)
    nxt = c + 2
    if nxt < nch:
      kk = nxt % NBUF
      if kk in pend_w:
        for cp in pend_w.pop(kk):
          cp.wait()
      pend_g[kk] = issue_gathers(kk, nxt)
  for cps in pend_w.values():
    for cp in cps:
      cp.wait()


@functools.partial(jax.jit, static_argnames=("nbatch", "seqlen", "dim"))
def _embed(sequence, token_table, pos_table, *, nbatch, seqlen, dim):
  mesh = plsc.VectorSubcoreMesh(core_axis_name="c", subcore_axis_name="s")
  kfn = pl.kernel(
      functools.partial(_embed_body, nbatch, seqlen, dim),
      mesh=mesh,
      out_type=jax.ShapeDtypeStruct((nbatch, seqlen, dim), jnp.float32),
      scratch_types=[
          pltpu.VMEM((nbatch, seqlen // NUM_WORKERS), jnp.int32),
          pltpu.VMEM((NBUF, CHUNK, dim), jnp.float32),
          pltpu.VMEM((NBUF, nbatch, CHUNK, dim), jnp.float32),
      ] + [pltpu.SemaphoreType.DMA] * (2 * NBUF),
  )
  return kfn(sequence, token_table, pos_table)


def kernel(sequence, token_table, pos_table):
  b, s = sequence.shape
  dim = token_table.shape[1]
  return _embed(sequence.astype(jnp.int32), token_table, pos_table,
                nbatch=b, seqlen=s, dim=dim)
